# Initial kernel scaffold; baseline (speedup 1.0000x reference)
import jax
import jax.numpy as jnp
from jax.experimental import pallas as pl


def kernel(embeddings, edge_index, Qw, Qb, Kw, Kb, Vw, Vb, W1, b1, W2, b2, W3, b3, Pw, Pb):
    raise NotImplementedError("write your pallas kernel here")


# speedup vs baseline: 1.1450x; 1.1450x over previous
import dataclasses
import functools

import jax
import jax.numpy as jnp
from jax import lax
from jax.experimental import pallas as pl
from jax.experimental.pallas import tpu as pltpu
from jax.experimental.pallas import tpu_sc as plsc

N = 512
E = 2048
D = 128
H = 64

_NUM_WORKERS = 32
_ROWS_PER_W = N // _NUM_WORKERS
_LANES = 16

_BI = 64


def _edge_scores_body(emb_ref, src_ref, dst_ref, qw_ref, qb_ref, kw_ref,
                      kb_ref, vw_ref, vb_ref, w1_ref, b1_ref, w2_ref, b2_ref,
                      w3_ref, b3_ref, pw_ref, sf_ref, sb_ref, vp_ref):
    f32 = jnp.float32
    emb = emb_ref[...]
    qe = jnp.dot(emb, qw_ref[...], preferred_element_type=f32) + qb_ref[...]
    ke = jnp.dot(emb, kw_ref[...], preferred_element_type=f32) + kb_ref[...]
    ve = jnp.dot(emb, vw_ref[...], preferred_element_type=f32) + vb_ref[...]
    vp_ref[...] = jnp.dot(ve, pw_ref[...], preferred_element_type=f32)

    col_iota = lax.broadcasted_iota(jnp.int32, (E, N), 1)
    oh_src = (src_ref[...] == col_iota).astype(f32)
    oh_dst = (dst_ref[...] == col_iota).astype(f32)
    qs = jnp.dot(oh_src, qe, preferred_element_type=f32)
    kd = jnp.dot(oh_dst, ke, preferred_element_type=f32)

    w1a = w1_ref[0:D, :]
    w1b = w1_ref[D:2 * D, :]

    def mlp(a, b):
        h = jnp.dot(a, w1a, preferred_element_type=f32)
        h = h + jnp.dot(b, w1b, preferred_element_type=f32) + b1_ref[...]
        h = jnp.maximum(h, 0.0)
        h = jnp.dot(h, w2_ref[...], preferred_element_type=f32) + b2_ref[...]
        h = jnp.maximum(h, 0.0)
        s = jnp.dot(h, w3_ref[...], preferred_element_type=f32) + b3_ref[...]
        return jnp.where(s >= 0.0, s, 0.2 * s)

    sf_ref[...] = mlp(qs, kd)
    sb_ref[...] = mlp(kd, qs)


def _edge_scores(emb, src_col, dst_col, qw, qb, kw, kb, vw, vb, w1, b1, w2,
                 b2, w3, b3, pw):
    f32 = jnp.float32
    return pl.pallas_call(
        _edge_scores_body,
        out_shape=[
            jax.ShapeDtypeStruct((E, 1), f32),
            jax.ShapeDtypeStruct((E, 1), f32),
            jax.ShapeDtypeStruct((N, D), f32),
        ],
    )(emb, src_col, dst_col, qw, qb, kw, kb, vw, vb, w1, b1, w2, b2, w3, b3,
      pw)


def _sc_scatter_body(src_hbm, dst_hbm, sf_hbm, sb_hbm, t_hbm, t_v, src_v,
                     dst_v, sf_v, sb_v):
    wid = lax.axis_index("s") * 2 + lax.axis_index("c")
    lo = wid * _ROWS_PER_W

    pltpu.sync_copy(src_hbm, src_v)
    pltpu.sync_copy(dst_hbm, dst_v)
    pltpu.sync_copy(sf_hbm, sf_v)
    pltpu.sync_copy(sb_hbm, sb_v)

    neg_inf = jnp.full((_LANES,), -jnp.inf, jnp.float32)

    @pl.loop(0, _ROWS_PER_W * N, step=_LANES)
    def _(k):
        t_v[pl.ds(k, _LANES)] = neg_inf

    @pl.loop(0, E, step=_LANES)
    def _(e):
        s16 = src_v[pl.ds(e, _LANES)]
        d16 = dst_v[pl.ds(e, _LANES)]
        v16 = sf_v[pl.ds(e, _LANES)]
        rel = s16 - lo
        m = (rel >= 0) & (rel < _ROWS_PER_W)
        idx = jnp.where(m, rel * N + d16, 0)
        plsc.store_scatter(t_v, [idx], v16, mask=m)

    @pl.loop(0, E, step=_LANES)
    def _(e):
        s16 = src_v[pl.ds(e, _LANES)]
        d16 = dst_v[pl.ds(e, _LANES)]
        v16 = sb_v[pl.ds(e, _LANES)]
        rel = d16 - lo
        m = (rel >= 0) & (rel < _ROWS_PER_W)
        idx = jnp.where(m, rel * N + s16, 0)
        plsc.store_scatter(t_v, [idx], v16, mask=m)

    pltpu.sync_copy(t_v, t_hbm.at[pl.ds(lo * N, _ROWS_PER_W * N)])


def _sc_scatter(src, dst, sf, sb):
    mesh = plsc.VectorSubcoreMesh(core_axis_name="c", subcore_axis_name="s")
    cp = pltpu.CompilerParams()
    if "needs_layout_passes" in pltpu.CompilerParams.__dataclass_fields__:
        cp = dataclasses.replace(cp, needs_layout_passes=False)
    return pl.kernel(
        _sc_scatter_body,
        out_type=jax.ShapeDtypeStruct((N * N,), jnp.float32),
        mesh=mesh,
        compiler_params=cp,
        scratch_types=[
            pltpu.VMEM((_ROWS_PER_W * N,), jnp.float32),
            pltpu.VMEM((E,), jnp.int32),
            pltpu.VMEM((E,), jnp.int32),
            pltpu.VMEM((E,), jnp.float32),
            pltpu.VMEM((E,), jnp.float32),
        ],
    )(src, dst, sf, sb)


def _softmax_outer_body(t_ref, vp_ref, pb_ref, out_ref):
    t = t_ref[...]
    m = jnp.max(t, axis=1, keepdims=True)
    e = jnp.exp(t - m)
    p = e / jnp.sum(e, axis=1, keepdims=True)
    vp = vp_ref[...]
    pb = pb_ref[...]
    out_ref[...] = p[:, :, None] * vp[None, :, :] + pb[None, :, :]


def _softmax_outer(t, vp, pb_row):
    return pl.pallas_call(
        _softmax_outer_body,
        grid=(N // _BI,),
        in_specs=[
            pl.BlockSpec((_BI, N), lambda i: (i, 0)),
            pl.BlockSpec((N, D), lambda i: (0, 0)),
            pl.BlockSpec((1, D), lambda i: (0, 0)),
        ],
        out_specs=pl.BlockSpec((_BI, N, D), lambda i: (i, 0, 0)),
        out_shape=jax.ShapeDtypeStruct((N, N, D), jnp.float32),
    )(t, vp, pb_row)


@jax.jit
def kernel(embeddings, edge_index, Qw, Qb, Kw, Kb, Vw, Vb, W1, b1, W2, b2,
           W3, b3, Pw, Pb):
    src = edge_index[0].astype(jnp.int32)
    dst = edge_index[1].astype(jnp.int32)

    sf, sb, vp = _edge_scores(
        embeddings, src.reshape(E, 1), dst.reshape(E, 1),
        Qw, Qb.reshape(1, D), Kw, Kb.reshape(1, D), Vw, Vb.reshape(1, D),
        W1, b1.reshape(1, H), W2, b2.reshape(1, H), W3, b3.reshape(1, 1), Pw)

    t_flat = _sc_scatter(src, dst, sf.reshape(E), sb.reshape(E))
    t = t_flat.reshape(N, N)

    return _softmax_outer(t, vp, Pb.reshape(1, D))
